# baseline (speedup 1.0000x reference)
import jax
import jax.numpy as jnp
from jax import lax
from jax.experimental import pallas as pl
from jax.experimental.pallas import tpu as pltpu
from jax.experimental.pallas import tpu_sc as plsc

M = 48
K = 256
DSUB = 16
B = 16384

NC = 2
NS = 16
NW = NC * NS
ROWS = B * M
RPW = ROWS // NW
DPW = B // NW
CHUNK = 1536
ISLICE = 128
NSTREAM = CHUNK // ISLICE
NCHUNK = RPW // CHUNK
DVC = CHUNK // (16 * M)


def _gather_body(codes_hbm, table_hbm, out_hbm, codes_v, idx2_v, rows0, rows1,
                 tab_s, sem_g0, sem_g1, sem_o0, sem_o1):
    wid = lax.axis_index("s") * NC + lax.axis_index("c")
    base = pl.multiple_of(wid * RPW, RPW)

    @pl.when(lax.axis_index("s") == 0)
    def _stage():
        pltpu.sync_copy(table_hbm, tab_s)

    plsc.subcore_barrier()

    pltpu.sync_copy(codes_hbm.at[pl.ds(base, RPW)], codes_v)

    lane = lax.iota(jnp.int32, 16)
    lane8 = lax.rem(lane, 8)
    lo = lane < 8

    def build_chunk(ci):
        def build(d8, carry):
            p0 = pl.multiple_of(d8 * 384, 16)
            a00 = d8 * (8 * M)
            for j in range(6):
                for q in range(4):
                    a0 = a00 + 2 * q * M + 8 * j
                    v1 = codes_v[pl.ds(pl.multiple_of(a0, 8), 16)]
                    v2 = codes_v[pl.ds(pl.multiple_of(a0 + M - 8, 8), 16)]
                    vals = jnp.where(lo, v1, v2) + (8 * j + lane8) * K
                    sl = pl.ds(pl.multiple_of(p0 + j * 64 + q * 16, 16), 16)
                    idx2_v[sl] = vals
            return carry

        lax.fori_loop(4 * ci, 4 * (ci + 1), build, 0)

    rows = (rows0, rows1)
    sem_g = (sem_g0, sem_g1)
    sem_o = (sem_o0, sem_o1)
    out_cp = [None, None]
    gth = [None, None]

    def drain(b, ci):
        for cp in gth[b]:
            cp.wait()
        gth[b] = None
        out_cp[b] = pltpu.async_copy(
            rows[b], out_hbm.at[pl.ds(base + ci * CHUNK, CHUNK)], sem_o[b]
        )

    for ci in range(NCHUNK):
        b = ci & 1
        if out_cp[b] is not None:
            out_cp[b].wait()
        build_chunk(ci)
        cb = ci * CHUNK
        gth[b] = [
            pltpu.async_copy(
                tab_s.at[idx2_v.at[pl.ds(cb + s * ISLICE, ISLICE)]],
                rows[b].at[pl.ds(s * ISLICE, ISLICE)],
                sem_g[b],
            )
            for s in range(NSTREAM)
        ]
        if ci >= 1:
            drain(1 - b, ci - 1)
    drain((NCHUNK - 1) & 1, NCHUNK - 1)
    out_cp[0].wait()
    out_cp[1].wait()


@jax.jit
def _impl(doc_codes, sub_weights):
    codes = doc_codes.astype(jnp.int32).reshape(ROWS)
    table = sub_weights.reshape(M * K, DSUB)
    mesh = plsc.VectorSubcoreMesh(core_axis_name="c", subcore_axis_name="s")
    out = pl.kernel(
        _gather_body,
        out_type=jax.ShapeDtypeStruct((ROWS, DSUB), jnp.float32),
        mesh=mesh,
        compiler_params=pltpu.CompilerParams(use_tc_tiling_on_sc=False),
        scratch_types=[
            pltpu.VMEM((RPW,), jnp.int32),
            pltpu.VMEM((RPW,), jnp.int32),
            pltpu.VMEM((CHUNK, DSUB), jnp.float32),
            pltpu.VMEM((CHUNK, DSUB), jnp.float32),
            pltpu.VMEM_SHARED((M * K, DSUB), jnp.float32),
            pltpu.SemaphoreType.DMA,
            pltpu.SemaphoreType.DMA,
            pltpu.SemaphoreType.DMA,
            pltpu.SemaphoreType.DMA,
        ],
    )(codes, table)
    out = out.reshape(B // 8, M // 8, 8, 128).transpose(0, 2, 1, 3)
    return out.reshape(B, M * DSUB)


def kernel(doc_codes, sub_weights):
    return _impl(doc_codes, sub_weights)


# speedup vs baseline: 1.0268x; 1.0268x over previous
import jax
import jax.numpy as jnp
from jax import lax
from jax.experimental import pallas as pl
from jax.experimental.pallas import tpu as pltpu
from jax.experimental.pallas import tpu_sc as plsc

M = 48
K = 256
DSUB = 16
B = 16384

NC = 2
NS = 16
NW = NC * NS
ROWS = B * M
RPW = ROWS // NW
CHUNK = 1536
ISLICE = 256
NSTREAM = CHUNK // ISLICE
NCHUNK = RPW // CHUNK


def _gather_body(codes_hbm, table_hbm, out_hbm, idx_v, idx2_v, rows0, rows1,
                 tab_s, sem_g0, sem_g1, sem_o0, sem_o1):
    wid = lax.axis_index("s") * NC + lax.axis_index("c")
    base = pl.multiple_of(wid * RPW, RPW)

    @pl.when(lax.axis_index("s") == 0)
    def _stage():
        pltpu.sync_copy(table_hbm, tab_s)

    plsc.subcore_barrier()

    pltpu.sync_copy(codes_hbm.at[pl.ds(base, RPW)], idx_v)

    lane = lax.iota(jnp.int32, 16)
    lane8 = lax.rem(lane, 8)
    lo = lane < 8
    def build(d8, carry):
        p0 = pl.multiple_of(d8 * 384, 16)
        a00 = d8 * (8 * M)
        for j in range(6):
            for q in range(4):
                a0 = a00 + 2 * q * M + 8 * j
                v1 = idx_v[pl.ds(pl.multiple_of(a0, 8), 16)]
                v2 = idx_v[pl.ds(pl.multiple_of(a0 + M - 8, 8), 16)]
                vals = jnp.where(lo, v1, v2)
                vals = vals + (8 * j + lane8) * K
                sl = pl.ds(pl.multiple_of(p0 + j * 64 + q * 16, 16), 16)
                idx2_v[sl] = vals
        return carry

    lax.fori_loop(0, RPW // 384, build, 0)

    rows = (rows0, rows1)
    sem_g = (sem_g0, sem_g1)
    sem_o = (sem_o0, sem_o1)
    out_cp = [None, None]
    gth = [None, None]

    def drain(b, ci):
        for cp in gth[b]:
            cp.wait()
        gth[b] = None
        out_cp[b] = pltpu.async_copy(
            rows[b], out_hbm.at[pl.ds(base + ci * CHUNK, CHUNK)], sem_o[b]
        )

    for ci in range(NCHUNK):
        b = ci & 1
        if out_cp[b] is not None:
            out_cp[b].wait()
        cb = ci * CHUNK
        gth[b] = [
            pltpu.async_copy(
                tab_s.at[idx2_v.at[pl.ds(cb + s * ISLICE, ISLICE)]],
                rows[b].at[pl.ds(s * ISLICE, ISLICE)],
                sem_g[b],
            )
            for s in range(NSTREAM)
        ]
        if ci >= 1:
            drain(1 - b, ci - 1)
    drain((NCHUNK - 1) & 1, NCHUNK - 1)
    out_cp[0].wait()
    out_cp[1].wait()


@jax.jit
def _impl(doc_codes, sub_weights):
    codes = doc_codes.astype(jnp.int32).reshape(ROWS)
    table = sub_weights.reshape(M * K, DSUB)
    mesh = plsc.VectorSubcoreMesh(core_axis_name="c", subcore_axis_name="s")
    out = pl.kernel(
        _gather_body,
        out_type=jax.ShapeDtypeStruct((ROWS, DSUB), jnp.float32),
        mesh=mesh,
        compiler_params=pltpu.CompilerParams(use_tc_tiling_on_sc=False),
        scratch_types=[
            pltpu.VMEM((RPW,), jnp.int32),
            pltpu.VMEM((RPW,), jnp.int32),
            pltpu.VMEM((CHUNK, DSUB), jnp.float32),
            pltpu.VMEM((CHUNK, DSUB), jnp.float32),
            pltpu.VMEM_SHARED((M * K, DSUB), jnp.float32),
            pltpu.SemaphoreType.DMA,
            pltpu.SemaphoreType.DMA,
            pltpu.SemaphoreType.DMA,
            pltpu.SemaphoreType.DMA,
        ],
    )(codes, table)
    out = out.reshape(B // 8, M // 8, 8, 128).transpose(0, 2, 1, 3)
    return out.reshape(B, M * DSUB)


def kernel(doc_codes, sub_weights):
    return _impl(doc_codes, sub_weights)
